# baseline (speedup 1.0000x reference)
import functools

import jax
import jax.numpy as jnp
from jax import lax
from jax.experimental import pallas as pl
from jax.experimental.pallas import tpu as pltpu
from jax.experimental.pallas import tpu_sc as plsc

_S = 64.0
_M0 = 0.62

_BN = 2000


def _gather_weights_sc(weights, label):
    (b,) = label.shape
    info = plsc.get_sparse_core_info()
    nw = info.num_cores * info.num_subcores
    b_per_w = b // nw
    mesh = plsc.VectorSubcoreMesh(core_axis_name="c", subcore_axis_name="s")

    @functools.partial(
        pl.kernel,
        mesh=mesh,
        out_type=jax.ShapeDtypeStruct((b,), jnp.float32),
        scratch_types=[
            pltpu.VMEM((b_per_w,), jnp.int32),
            pltpu.VMEM((b_per_w,), jnp.float32),
            pltpu.SemaphoreType.DMA,
        ],
    )
    def gather_kernel(weights_hbm, label_hbm, out_hbm, idx_v, vals_v, sem):
        wid = lax.axis_index("s") * info.num_cores + lax.axis_index("c")
        base = wid * b_per_w
        pltpu.sync_copy(label_hbm.at[pl.ds(base, b_per_w)], idx_v)
        pltpu.async_copy(weights_hbm.at[idx_v], vals_v, sem).wait()
        pltpu.sync_copy(vals_v, out_hbm.at[pl.ds(base, b_per_w)])

    return gather_kernel(weights, label)


def _dense_body(label_ref, wlab_ref, cos_ref, out_ref):
    j = pl.program_id(0)
    c = cos_ref[...]
    lab = label_ref[...]
    row = lax.broadcasted_iota(jnp.int32, c.shape, 0) + j * _BN
    mask = lab == row
    c_lab = jnp.sum(jnp.where(mask, c, 0.0), axis=0, keepdims=True)
    d = _M0 * wlab_ref[...]
    sin_theta = jnp.sqrt(jnp.maximum(1.0 - c_lab * c_lab, 0.0))
    fix = (c_lab * jnp.cos(d) - sin_theta * jnp.sin(d)) * _S
    out_ref[...] = jnp.where(mask, fix, c * _S)


def kernel(cosine, label, weights):
    b, n_cols = cosine.shape
    w_lab = _gather_weights_sc(weights, label)
    ct = cosine.T
    out_t = pl.pallas_call(
        _dense_body,
        grid=(pl.cdiv(n_cols, _BN),),
        in_specs=[
            pl.BlockSpec((1, b), lambda j: (0, 0)),
            pl.BlockSpec((1, b), lambda j: (0, 0)),
            pl.BlockSpec((_BN, b), lambda j: (j, 0)),
        ],
        out_specs=pl.BlockSpec((_BN, b), lambda j: (j, 0)),
        out_shape=jax.ShapeDtypeStruct((n_cols, b), jnp.float32),
    )(label.reshape(1, b), w_lab.reshape(1, b), ct)
    return out_t.T


# speedup vs baseline: 1.0051x; 1.0051x over previous
import functools

import jax
import jax.numpy as jnp
from jax import lax
from jax.experimental import pallas as pl
from jax.experimental.pallas import tpu as pltpu
from jax.experimental.pallas import tpu_sc as plsc

_S = 64.0
_M0 = 0.62

_BN = 3200


def _gather_weights_sc(weights, label):
    (b,) = label.shape
    info = plsc.get_sparse_core_info()
    nw = info.num_cores * info.num_subcores
    b_per_w = b // nw
    mesh = plsc.VectorSubcoreMesh(core_axis_name="c", subcore_axis_name="s")

    @functools.partial(
        pl.kernel,
        mesh=mesh,
        out_type=jax.ShapeDtypeStruct((b,), jnp.float32),
        scratch_types=[
            pltpu.VMEM((b_per_w,), jnp.int32),
            pltpu.VMEM((b_per_w,), jnp.float32),
            pltpu.SemaphoreType.DMA,
        ],
    )
    def gather_kernel(weights_hbm, label_hbm, out_hbm, idx_v, vals_v, sem):
        wid = lax.axis_index("s") * info.num_cores + lax.axis_index("c")
        base = wid * b_per_w
        pltpu.sync_copy(label_hbm.at[pl.ds(base, b_per_w)], idx_v)
        pltpu.async_copy(weights_hbm.at[idx_v], vals_v, sem).wait()
        pltpu.sync_copy(vals_v, out_hbm.at[pl.ds(base, b_per_w)])

    return gather_kernel(weights, label)


def _dense_body(label_ref, wlab_ref, cos_ref, out_ref):
    j = pl.program_id(0)
    c = cos_ref[...]
    lab = label_ref[...]
    row = lax.broadcasted_iota(jnp.int32, c.shape, 0) + j * _BN
    mask = lab == row
    c_lab = jnp.sum(jnp.where(mask, c, 0.0), axis=0, keepdims=True)
    d = _M0 * wlab_ref[...]
    sin_theta = jnp.sqrt(jnp.maximum(1.0 - c_lab * c_lab, 0.0))
    fix = (c_lab * jnp.cos(d) - sin_theta * jnp.sin(d)) * _S
    out_ref[...] = jnp.where(mask, fix, c * _S)


def kernel(cosine, label, weights):
    b, n_cols = cosine.shape
    w_lab = _gather_weights_sc(weights, label)
    ct = cosine.T
    out_t = pl.pallas_call(
        _dense_body,
        grid=(pl.cdiv(n_cols, _BN),),
        in_specs=[
            pl.BlockSpec((1, b), lambda j: (0, 0)),
            pl.BlockSpec((1, b), lambda j: (0, 0)),
            pl.BlockSpec((_BN, b), lambda j: (j, 0)),
        ],
        out_specs=pl.BlockSpec((_BN, b), lambda j: (j, 0)),
        out_shape=jax.ShapeDtypeStruct((n_cols, b), jnp.float32),
    )(label.reshape(1, b), w_lab.reshape(1, b), ct)
    return out_t.T


# speedup vs baseline: 1.0117x; 1.0065x over previous
import functools

import jax
import jax.numpy as jnp
from jax import lax
from jax.experimental import pallas as pl
from jax.experimental.pallas import tpu as pltpu
from jax.experimental.pallas import tpu_sc as plsc

_S = 64.0
_M0 = 0.62

_BN = 3200


def _gather_weights_sc(weights, label):
    (b,) = label.shape
    info = plsc.get_sparse_core_info()
    nc = 1
    nw = nc * info.num_subcores
    b_per_w = b // nw
    mesh = plsc.VectorSubcoreMesh(
        core_axis_name="c", subcore_axis_name="s", num_cores=nc
    )

    @functools.partial(
        pl.kernel,
        mesh=mesh,
        out_type=jax.ShapeDtypeStruct((b,), jnp.float32),
        scratch_types=[
            pltpu.VMEM((b_per_w,), jnp.int32),
            pltpu.VMEM((b_per_w,), jnp.float32),
            pltpu.SemaphoreType.DMA,
        ],
    )
    def gather_kernel(weights_hbm, label_hbm, out_hbm, idx_v, vals_v, sem):
        wid = lax.axis_index("s") * nc + lax.axis_index("c")
        base = wid * b_per_w
        pltpu.sync_copy(label_hbm.at[pl.ds(base, b_per_w)], idx_v)
        pltpu.async_copy(weights_hbm.at[idx_v], vals_v, sem).wait()
        pltpu.sync_copy(vals_v, out_hbm.at[pl.ds(base, b_per_w)])

    return gather_kernel(weights, label)


def _dense_body(label_ref, wlab_ref, cos_ref, out_ref):
    j = pl.program_id(0)
    c = cos_ref[...]
    lab = label_ref[...]
    row = lax.broadcasted_iota(jnp.int32, c.shape, 0) + j * _BN
    mask = lab == row
    c_lab = jnp.sum(jnp.where(mask, c, 0.0), axis=0, keepdims=True)
    d = _M0 * wlab_ref[...]
    sin_theta = jnp.sqrt(jnp.maximum(1.0 - c_lab * c_lab, 0.0))
    fix = (c_lab * jnp.cos(d) - sin_theta * jnp.sin(d)) * _S
    out_ref[...] = jnp.where(mask, fix, c * _S)


def kernel(cosine, label, weights):
    b, n_cols = cosine.shape
    w_lab = _gather_weights_sc(weights, label)
    ct = cosine.T
    out_t = pl.pallas_call(
        _dense_body,
        grid=(pl.cdiv(n_cols, _BN),),
        in_specs=[
            pl.BlockSpec((1, b), lambda j: (0, 0)),
            pl.BlockSpec((1, b), lambda j: (0, 0)),
            pl.BlockSpec((_BN, b), lambda j: (j, 0)),
        ],
        out_specs=pl.BlockSpec((_BN, b), lambda j: (j, 0)),
        out_shape=jax.ShapeDtypeStruct((n_cols, b), jnp.float32),
    )(label.reshape(1, b), w_lab.reshape(1, b), ct)
    return out_t.T
